# baseline (speedup 1.0000x reference)
import functools

import numpy as np
import jax
import jax.numpy as jnp
from jax import lax
from jax.experimental import pallas as pl
from jax.experimental.pallas import tpu as pltpu
from jax.experimental.pallas import tpu_sc as plsc

LEVELS = 8
L_NH = 6
FEAT = 4
LOG2 = 19
TABLE = 1 << LOG2
RES = (16, 23, 32, 45, 64, 80, 128, 256)
PI2 = np.int32(-1640531535)
PI3 = np.int32(805459861)
MASK = np.int32(TABLE - 1)

CORNERS = [(a, b, c) for a in (0, 1) for b in (0, 1) for c in (0, 1)]

NC, NS = 2, 16
NW = NC * NS
GCH = 128
KPC = 16
CH = KPC * GCH


def _index_body(xt_ref, gidx_ref, w_ref):
    x = xt_ref[0]
    y = xt_ref[1]
    t = xt_ref[2]
    for l in range(LEVELS):
        r = float(RES[l])
        sx = x * r
        sy = y * r
        st = t * r
        fx = sx.astype(jnp.int32)
        fy = sy.astype(jnp.int32)
        ft = st.astype(jnp.int32)
        hx = sx - fx.astype(jnp.float32)
        hy = sy - fy.astype(jnp.float32)
        ht = st - ft.astype(jnp.float32)
        wx = (1.0 - hx, hx)
        wy = (1.0 - hy, hy)
        wt = (1.0 - ht, ht)
        wxy = [[wx[a] * wy[b] for b in (0, 1)] for a in (0, 1)]
        base = np.int32(l * TABLE)
        if l < L_NH:
            r1 = np.int32(RES[l])
            r2 = np.int32(RES[l] * RES[l])
            lin = fx + fy * r1 + ft * r2 + base
            offs = [np.int32(a + b * int(r1) + c * int(r2))
                    for (a, b, c) in CORNERS]
            idxs = [lin + o for o in offs]
        else:
            gx = (fx, fx + 1)
            gy = (fy * PI2, fy * PI2 + PI2)
            gt = (ft * PI3, ft * PI3 + PI3)
            gxy = [[gx[a] ^ gy[b] for b in (0, 1)] for a in (0, 1)]
            idxs = [((gxy[a][b] ^ gt[c]) & MASK) + base
                    for (a, b, c) in CORNERS]
        for ci, (a, b, c) in enumerate(CORNERS):
            gidx_ref[l * 8 + ci] = idxs[ci]
            w_ref[l * 8 + ci] = wxy[a][b] * wt[c]


def _index_call(xt):
    rows = xt.shape[1]
    sb = 64
    return pl.pallas_call(
        _index_body,
        grid=(rows // sb,),
        in_specs=[pl.BlockSpec((3, sb, 128), lambda i: (0, i, 0))],
        out_specs=[pl.BlockSpec((64, sb, 128), lambda i: (0, i, 0)),
                   pl.BlockSpec((64, sb, 128), lambda i: (0, i, 0))],
        out_shape=[jax.ShapeDtypeStruct((64, rows, 128), jnp.int32),
                   jax.ShapeDtypeStruct((64, rows, 128), jnp.float32)],
    )(xt)


def _gather_call(table8, idx2d):
    tot = idx2d.shape[0] * 128
    npw = tot // NW
    niter = npw // CH
    mesh = plsc.VectorSubcoreMesh(core_axis_name="c", subcore_axis_name="s")

    @functools.partial(
        pl.kernel,
        out_type=jax.ShapeDtypeStruct((tot, FEAT), jnp.float32),
        mesh=mesh,
        compiler_params=pltpu.CompilerParams(use_tc_tiling_on_sc=False),
        scratch_types=[
            pltpu.VMEM((KPC, GCH), jnp.int32),
            pltpu.VMEM((CH, 2 * FEAT), jnp.float32),
            pltpu.SemaphoreType.DMA,
        ],
    )
    def gk(table_hbm, idx_hbm, out_hbm, idx_v, rows_v, sem):
        wid = lax.axis_index("s") * NC + lax.axis_index("c")
        base = wid * npw

        @pl.loop(0, niter)
        def _(it):
            off = pl.multiple_of(base + it * CH, CH)
            irow = pl.multiple_of(off // 128, KPC)
            pltpu.sync_copy(idx_hbm.at[pl.ds(irow, KPC)], idx_v)
            copies = [
                pltpu.async_copy(
                    table_hbm.at[idx_v.at[k]],
                    rows_v.at[pl.ds(k * GCH, GCH)],
                    sem,
                )
                for k in range(KPC)
            ]
            for cp in copies:
                cp.wait()
            pltpu.sync_copy(rows_v.at[:, pl.ds(0, FEAT)],
                            out_hbm.at[pl.ds(off, CH)])

    return gk(table8, idx2d)


def _reduce_body(f_ref, w_ref, e_ref, s_ref, o_ref):
    w256 = jnp.dot(w_ref[...], e_ref[...],
                   preferred_element_type=jnp.float32,
                   precision=lax.Precision.HIGHEST)
    p = f_ref[...] * w256
    o_ref[...] = jnp.dot(p, s_ref[...],
                         preferred_element_type=jnp.float32,
                         precision=lax.Precision.HIGHEST)


def _reduce_call(feats2, w_pm, e_mat, s_mat):
    bn = feats2.shape[0]
    nt = 1024
    return pl.pallas_call(
        _reduce_body,
        grid=(bn // nt,),
        in_specs=[pl.BlockSpec((nt, 256), lambda i: (i, 0)),
                  pl.BlockSpec((nt, 64), lambda i: (i, 0)),
                  pl.BlockSpec((64, 256), lambda i: (0, 0)),
                  pl.BlockSpec((256, 32), lambda i: (0, 0))],
        out_specs=pl.BlockSpec((nt, 32), lambda i: (i, 0)),
        out_shape=jax.ShapeDtypeStruct((bn, 32), jnp.float32),
    )(feats2, w_pm, e_mat, s_mat)


def _const_mats():
    e = np.repeat(np.eye(64, dtype=np.float32), FEAT, axis=1)
    s = np.kron(np.eye(LEVELS, dtype=np.float32),
                np.kron(np.ones((8, 1), np.float32),
                        np.eye(FEAT, dtype=np.float32)))
    return jnp.asarray(e), jnp.asarray(s)


def kernel(eventBlock, hashmap):
    b, n, _ = eventBlock.shape
    bn = b * n
    xt = eventBlock.reshape(bn, 3).T.reshape(3, bn // 128, 128)
    gidx_cm, w_cm = _index_call(xt)
    gidx_pm = gidx_cm.reshape(64, bn).T.reshape((bn * 64) // 128, 128)
    w_pm = w_cm.reshape(64, bn).T
    table8 = jnp.pad(hashmap.reshape(LEVELS * TABLE, FEAT),
                     ((0, 0), (0, FEAT)))
    feats = _gather_call(table8, gidx_pm)
    e_mat, s_mat = _const_mats()
    out = _reduce_call(feats.reshape(bn, 64 * FEAT), w_pm, e_mat, s_mat)
    return out.reshape(b, n, LEVELS * FEAT)


# speedup vs baseline: 3.8453x; 3.8453x over previous
import functools

import numpy as np
import jax
import jax.numpy as jnp
from jax import lax
from jax.experimental import pallas as pl
from jax.experimental.pallas import tpu as pltpu
from jax.experimental.pallas import tpu_sc as plsc

LEVELS = 8
L_NH = 6
FEAT = 4
LOG2 = 19
TABLE = 1 << LOG2
RES = (16, 23, 32, 45, 64, 80, 128, 256)
PI2 = np.int32(-1640531535)
PI3 = np.int32(805459861)
MASK = np.int32(TABLE - 1)

CORNERS = [(a, b, c) for a in (0, 1) for b in (0, 1) for c in (0, 1)]

NC, NS = 2, 16
NW = NC * NS
GCH = 2048


def _index_body(xt_ref, gidx_ref, w_ref):
    x = xt_ref[0]
    y = xt_ref[1]
    t = xt_ref[2]
    for l in range(LEVELS):
        r = float(RES[l])
        sx = x * r
        sy = y * r
        st = t * r
        fx = sx.astype(jnp.int32)
        fy = sy.astype(jnp.int32)
        ft = st.astype(jnp.int32)
        hx = sx - fx.astype(jnp.float32)
        hy = sy - fy.astype(jnp.float32)
        ht = st - ft.astype(jnp.float32)
        wx = (1.0 - hx, hx)
        wy = (1.0 - hy, hy)
        wt = (1.0 - ht, ht)
        wxy = [[wx[a] * wy[b] for b in (0, 1)] for a in (0, 1)]
        base = np.int32(l * TABLE)
        if l < L_NH:
            r1 = np.int32(RES[l])
            r2 = np.int32(RES[l] * RES[l])
            lin = fx + fy * r1 + ft * r2 + base
            offs = [np.int32(a + b * int(r1) + c * int(r2))
                    for (a, b, c) in CORNERS]
            idxs = [lin + o for o in offs]
        else:
            gx = (fx, fx + 1)
            gy = (fy * PI2, fy * PI2 + PI2)
            gt = (ft * PI3, ft * PI3 + PI3)
            gxy = [[gx[a] ^ gy[b] for b in (0, 1)] for a in (0, 1)]
            idxs = [((gxy[a][b] ^ gt[c]) & MASK) + base
                    for (a, b, c) in CORNERS]
        for ci, (a, b, c) in enumerate(CORNERS):
            full = idxs[ci]
            par = full & 1
            w = wxy[a][b] * wt[c]
            gidx_ref[l * 8 + ci] = lax.shift_right_logical(full, 1)
            w_ref[l * 8 + ci] = jnp.where(par == 1, -w, w)


def _index_call(xt):
    rows = xt.shape[1]
    sb = 64
    return pl.pallas_call(
        _index_body,
        grid=(rows // sb,),
        in_specs=[pl.BlockSpec((3, sb, 128), lambda i: (0, i, 0))],
        out_specs=[pl.BlockSpec((64, sb, 128), lambda i: (0, i, 0)),
                   pl.BlockSpec((64, sb, 128), lambda i: (0, i, 0))],
        out_shape=[jax.ShapeDtypeStruct((64, rows, 128), jnp.int32),
                   jax.ShapeDtypeStruct((64, rows, 128), jnp.float32)],
    )(xt)


def _tr_body(i_ref, w_ref, oi_ref, ow_ref):
    oi_ref[...] = i_ref[...].T
    ow_ref[...] = w_ref[...].T


def _transpose_call(gidx_cm, w_cm):
    bn = gidx_cm.shape[1]
    nt = 512
    return pl.pallas_call(
        _tr_body,
        grid=(bn // nt,),
        in_specs=[pl.BlockSpec((64, nt), lambda i: (0, i)),
                  pl.BlockSpec((64, nt), lambda i: (0, i))],
        out_specs=[pl.BlockSpec((nt, 64), lambda i: (i, 0)),
                   pl.BlockSpec((nt, 64), lambda i: (i, 0))],
        out_shape=[jax.ShapeDtypeStruct((bn, 64), jnp.int32),
                   jax.ShapeDtypeStruct((bn, 64), jnp.float32)],
    )(gidx_cm, w_cm)


def _gather_call(pairs, idx2d):
    tot = idx2d.shape[0] * GCH
    npw = tot // NW
    niter = npw // GCH
    mesh = plsc.VectorSubcoreMesh(core_axis_name="c", subcore_axis_name="s")

    @functools.partial(
        pl.kernel,
        out_type=jax.ShapeDtypeStruct((tot, 8), jnp.float32),
        mesh=mesh,
        compiler_params=pltpu.CompilerParams(use_tc_tiling_on_sc=False),
        scratch_types=[
            pltpu.VMEM((GCH,), jnp.int32),
            pltpu.VMEM((GCH, 8), jnp.float32),
            pltpu.SemaphoreType.DMA,
        ],
    )
    def gk(pairs_hbm, idx_hbm, out_hbm, idx_v, rows_v, sem):
        wid = lax.axis_index("s") * NC + lax.axis_index("c")
        base = wid * npw

        @pl.loop(0, niter)
        def _(it):
            off = pl.multiple_of(base + it * GCH, GCH)
            pltpu.sync_copy(idx_hbm.at[off // GCH], idx_v)
            pltpu.async_copy(pairs_hbm.at[idx_v], rows_v, sem).wait()
            pltpu.sync_copy(rows_v, out_hbm.at[pl.ds(off, GCH)])

    return gk(pairs, idx2d)


def _reduce_body(f_ref, w_ref, elo_ref, ehi_ref, s_ref, o_ref):
    ws = w_ref[...]
    wlo = jnp.maximum(ws, 0.0)
    whi = jnp.maximum(-ws, 0.0)
    w512 = (jnp.dot(wlo, elo_ref[...], preferred_element_type=jnp.float32,
                    precision=lax.Precision.HIGHEST)
            + jnp.dot(whi, ehi_ref[...], preferred_element_type=jnp.float32,
                      precision=lax.Precision.HIGHEST))
    p = f_ref[...] * w512
    o_ref[...] = jnp.dot(p, s_ref[...], preferred_element_type=jnp.float32,
                         precision=lax.Precision.HIGHEST)


def _reduce_call(feats2, w_pm, elo, ehi, s_mat):
    bn = feats2.shape[0]
    nt = 512
    return pl.pallas_call(
        _reduce_body,
        grid=(bn // nt,),
        in_specs=[pl.BlockSpec((nt, 512), lambda i: (i, 0)),
                  pl.BlockSpec((nt, 64), lambda i: (i, 0)),
                  pl.BlockSpec((64, 512), lambda i: (0, 0)),
                  pl.BlockSpec((64, 512), lambda i: (0, 0)),
                  pl.BlockSpec((512, 32), lambda i: (0, 0))],
        out_specs=pl.BlockSpec((nt, 32), lambda i: (i, 0)),
        out_shape=jax.ShapeDtypeStruct((bn, 32), jnp.float32),
    )(feats2, w_pm, elo, ehi, s_mat)


def _const_mats():
    elo = np.zeros((64, 512), np.float32)
    ehi = np.zeros((64, 512), np.float32)
    for c in range(64):
        for f in range(FEAT):
            elo[c, c * 8 + f] = 1.0
            ehi[c, c * 8 + 4 + f] = 1.0
    s = np.kron(np.eye(LEVELS, dtype=np.float32),
                np.kron(np.ones((8, 1), np.float32),
                        np.vstack([np.eye(FEAT, dtype=np.float32),
                                   np.eye(FEAT, dtype=np.float32)])))
    return jnp.asarray(elo), jnp.asarray(ehi), jnp.asarray(s)


def kernel(eventBlock, hashmap):
    b, n, _ = eventBlock.shape
    bn = b * n
    xt = eventBlock.reshape(bn, 3).T.reshape(3, bn // 128, 128)
    gidx_cm, w_cm = _index_call(xt)
    gidx_pm, w_pm = _transpose_call(gidx_cm.reshape(64, bn),
                                    w_cm.reshape(64, bn))
    pairs = hashmap.reshape(LEVELS * TABLE // 2, 2 * FEAT)
    feats = _gather_call(pairs, gidx_pm.reshape((bn * 64) // GCH, GCH))
    elo, ehi, s_mat = _const_mats()
    out = _reduce_call(feats.reshape(bn, 512), w_pm, elo, ehi, s_mat)
    return out.reshape(b, n, LEVELS * FEAT)


# speedup vs baseline: 3.9083x; 1.0164x over previous
import functools

import numpy as np
import jax
import jax.numpy as jnp
from jax import lax
from jax.experimental import pallas as pl
from jax.experimental.pallas import tpu as pltpu
from jax.experimental.pallas import tpu_sc as plsc

LEVELS = 8
L_NH = 6
FEAT = 4
LOG2 = 19
TABLE = 1 << LOG2
RES = (16, 23, 32, 45, 64, 80, 128, 256)
PI2 = np.int32(-1640531535)
PI3 = np.int32(805459861)
MASK = np.int32(TABLE - 1)

CORNERS = [(a, b, c) for a in (0, 1) for b in (0, 1) for c in (0, 1)]

NC, NS = 2, 16
NW = NC * NS
GCH = 2048


def _index_body(xt_ref, gidx_ref, w_ref):
    x = xt_ref[0]
    y = xt_ref[1]
    t = xt_ref[2]
    for l in range(LEVELS):
        r = float(RES[l])
        sx = x * r
        sy = y * r
        st = t * r
        fx = sx.astype(jnp.int32)
        fy = sy.astype(jnp.int32)
        ft = st.astype(jnp.int32)
        hx = sx - fx.astype(jnp.float32)
        hy = sy - fy.astype(jnp.float32)
        ht = st - ft.astype(jnp.float32)
        wx = (1.0 - hx, hx)
        wy = (1.0 - hy, hy)
        wt = (1.0 - ht, ht)
        wxy = [[wx[a] * wy[b] for b in (0, 1)] for a in (0, 1)]
        base = np.int32(l * TABLE)
        if l < L_NH:
            r1 = np.int32(RES[l])
            r2 = np.int32(RES[l] * RES[l])
            lin = fx + fy * r1 + ft * r2 + base
            offs = [np.int32(a + b * int(r1) + c * int(r2))
                    for (a, b, c) in CORNERS]
            idxs = [lin + o for o in offs]
        else:
            gx = (fx, fx + 1)
            gy = (fy * PI2, fy * PI2 + PI2)
            gt = (ft * PI3, ft * PI3 + PI3)
            gxy = [[gx[a] ^ gy[b] for b in (0, 1)] for a in (0, 1)]
            idxs = [((gxy[a][b] ^ gt[c]) & MASK) + base
                    for (a, b, c) in CORNERS]
        for ci, (a, b, c) in enumerate(CORNERS):
            full = idxs[ci]
            par = full & 1
            w = wxy[a][b] * wt[c]
            gidx_ref[l * 8 + ci] = (
                lax.shift_right_logical(full, 1).astype(jnp.float32))
            w_ref[l * 8 + ci] = jnp.where(par == 1, -w, w)


def _index_call(xt):
    rows = xt.shape[1]
    sb = 64
    return pl.pallas_call(
        _index_body,
        grid=(rows // sb,),
        in_specs=[pl.BlockSpec((3, sb, 128), lambda i: (0, i, 0))],
        out_specs=[pl.BlockSpec((64, sb, 128), lambda i: (0, i, 0)),
                   pl.BlockSpec((64, sb, 128), lambda i: (0, i, 0))],
        out_shape=[jax.ShapeDtypeStruct((64, rows, 128), jnp.float32),
                   jax.ShapeDtypeStruct((64, rows, 128), jnp.float32)],
    )(xt)


def _tr_body(a0_ref, a1_ref, i_ref, oi_ref):
    a0 = a0_ref[...]
    a1 = a1_ref[...]
    parts = []
    for j in range(8):
        t = i_ref[:, j, :].T
        lo = jnp.dot(a0, t, preferred_element_type=jnp.float32,
                     precision=lax.Precision.HIGHEST)
        hi = jnp.dot(a1, t, preferred_element_type=jnp.float32,
                     precision=lax.Precision.HIGHEST)
        parts.append(jnp.concatenate([lo, hi], axis=1))
    oi_ref[...] = jnp.concatenate(parts, axis=0).astype(jnp.int32)


def _transpose_call(gidx_cm3):
    rows = gidx_cm3.shape[1]
    bn = rows * 128
    a0 = np.zeros((64, 128), np.float32)
    a1 = np.zeros((64, 128), np.float32)
    for u in range(64):
        a0[u, 2 * u] = 1.0
        a1[u, 2 * u + 1] = 1.0
    return pl.pallas_call(
        _tr_body,
        grid=(bn // 1024,),
        in_specs=[pl.BlockSpec((64, 128), lambda i: (0, 0)),
                  pl.BlockSpec((64, 128), lambda i: (0, 0)),
                  pl.BlockSpec((64, 8, 128), lambda i: (0, i, 0))],
        out_specs=pl.BlockSpec((512, 128), lambda i: (i, 0)),
        out_shape=jax.ShapeDtypeStruct((bn * 64 // 128, 128), jnp.int32),
    )(jnp.asarray(a0), jnp.asarray(a1), gidx_cm3)


def _gather_call(pairs, idx2d):
    tot = idx2d.shape[0] * 128
    npw = tot // NW
    niter = npw // GCH
    mesh = plsc.VectorSubcoreMesh(core_axis_name="c", subcore_axis_name="s")

    @functools.partial(
        pl.kernel,
        out_type=jax.ShapeDtypeStruct((tot, 8), jnp.float32),
        mesh=mesh,
        compiler_params=pltpu.CompilerParams(use_tc_tiling_on_sc=False),
        scratch_types=[
            pltpu.VMEM((GCH // 128, 128), jnp.int32),
            pltpu.VMEM((GCH, 8), jnp.float32),
            pltpu.SemaphoreType.DMA,
        ],
    )
    def gk(pairs_hbm, idx_hbm, out_hbm, idx_v, rows_v, sem):
        wid = lax.axis_index("s") * NC + lax.axis_index("c")
        base = wid * npw
        kpc = GCH // 128

        @pl.loop(0, niter)
        def _(it):
            off = pl.multiple_of(base + it * GCH, GCH)
            irow = pl.multiple_of(off // 128, kpc)
            pltpu.sync_copy(idx_hbm.at[pl.ds(irow, kpc)], idx_v)
            copies = [
                pltpu.async_copy(pairs_hbm.at[idx_v.at[k]],
                                 rows_v.at[pl.ds(k * 128, 128)], sem)
                for k in range(kpc)
            ]
            for cp in copies:
                cp.wait()
            pltpu.sync_copy(rows_v, out_hbm.at[pl.ds(off, GCH)])

    return gk(pairs, idx2d)


def _reduce_body(f_ref, w_ref, elo_ref, ehi_ref, s_ref, o_ref):
    ws = jnp.concatenate([w_ref[:, j, :].T for j in range(8)], axis=0)
    wlo = jnp.maximum(ws, 0.0)
    whi = jnp.maximum(-ws, 0.0)
    w512 = (jnp.dot(wlo, elo_ref[...], preferred_element_type=jnp.float32,
                    precision=lax.Precision.HIGHEST)
            + jnp.dot(whi, ehi_ref[...], preferred_element_type=jnp.float32,
                      precision=lax.Precision.HIGHEST))
    p = f_ref[...] * w512
    o_ref[...] = jnp.dot(p, s_ref[...], preferred_element_type=jnp.float32,
                         precision=lax.Precision.HIGHEST)


def _reduce_call(feats2, w_pm, elo, ehi, s_mat):
    bn = feats2.shape[0]
    nt = 1024
    return pl.pallas_call(
        _reduce_body,
        grid=(bn // nt,),
        in_specs=[pl.BlockSpec((nt, 512), lambda i: (i, 0)),
                  pl.BlockSpec((64, 8, 128), lambda i: (0, i, 0)),
                  pl.BlockSpec((64, 512), lambda i: (0, 0)),
                  pl.BlockSpec((64, 512), lambda i: (0, 0)),
                  pl.BlockSpec((512, 32), lambda i: (0, 0))],
        out_specs=pl.BlockSpec((nt, 32), lambda i: (i, 0)),
        out_shape=jax.ShapeDtypeStruct((bn, 32), jnp.float32),
    )(feats2, w_pm, elo, ehi, s_mat)


def _const_mats():
    elo = np.zeros((64, 512), np.float32)
    ehi = np.zeros((64, 512), np.float32)
    for c in range(64):
        for f in range(FEAT):
            elo[c, c * 8 + f] = 1.0
            ehi[c, c * 8 + 4 + f] = 1.0
    s = np.kron(np.eye(LEVELS, dtype=np.float32),
                np.kron(np.ones((8, 1), np.float32),
                        np.vstack([np.eye(FEAT, dtype=np.float32),
                                   np.eye(FEAT, dtype=np.float32)])))
    return jnp.asarray(elo), jnp.asarray(ehi), jnp.asarray(s)


def kernel(eventBlock, hashmap):
    b, n, _ = eventBlock.shape
    bn = b * n
    xt = eventBlock.reshape(bn, 3).T.reshape(3, bn // 128, 128)
    gidx_cm3, w_cm3 = _index_call(xt)
    ridx = _transpose_call(gidx_cm3)
    pairs = hashmap.reshape(LEVELS * TABLE // 2, 2 * FEAT)
    feats = _gather_call(pairs, ridx)
    elo, ehi, s_mat = _const_mats()
    out = _reduce_call(feats.reshape(bn, 512), w_cm3, elo, ehi, s_mat)
    return out.reshape(b, n, LEVELS * FEAT)


# speedup vs baseline: 4.2856x; 1.0965x over previous
import functools

import numpy as np
import jax
import jax.numpy as jnp
from jax import lax
from jax.experimental import pallas as pl
from jax.experimental.pallas import tpu as pltpu
from jax.experimental.pallas import tpu_sc as plsc

LEVELS = 8
L_NH = 6
FEAT = 4
LOG2 = 19
TABLE = 1 << LOG2
RES = (16, 23, 32, 45, 64, 80, 128, 256)
PI2 = np.int32(-1640531535)
PI3 = np.int32(805459861)
MASK = np.int32(TABLE - 1)

CORNERS = [(a, b, c) for a in (0, 1) for b in (0, 1) for c in (0, 1)]

NC, NS = 2, 16
NW = NC * NS
GCH = 2048


def _index_body(xt_ref, gidx_ref, w_ref):
    x = xt_ref[0]
    y = xt_ref[1]
    t = xt_ref[2]
    for l in range(LEVELS):
        r = float(RES[l])
        sx = x * r
        sy = y * r
        st = t * r
        fx = sx.astype(jnp.int32)
        fy = sy.astype(jnp.int32)
        ft = st.astype(jnp.int32)
        hx = sx - fx.astype(jnp.float32)
        hy = sy - fy.astype(jnp.float32)
        ht = st - ft.astype(jnp.float32)
        wx = (1.0 - hx, hx)
        wy = (1.0 - hy, hy)
        wt = (1.0 - ht, ht)
        wxy = [[wx[a] * wy[b] for b in (0, 1)] for a in (0, 1)]
        base = np.int32(l * TABLE)
        if l < L_NH:
            r1 = np.int32(RES[l])
            r2 = np.int32(RES[l] * RES[l])
            lin = fx + fy * r1 + ft * r2 + base
            offs = [np.int32(a + b * int(r1) + c * int(r2))
                    for (a, b, c) in CORNERS]
            idxs = [lin + o for o in offs]
        else:
            gx = (fx, fx + 1)
            gy = (fy * PI2, fy * PI2 + PI2)
            gt = (ft * PI3, ft * PI3 + PI3)
            gxy = [[gx[a] ^ gy[b] for b in (0, 1)] for a in (0, 1)]
            idxs = [((gxy[a][b] ^ gt[c]) & MASK) + base
                    for (a, b, c) in CORNERS]
        for ci, (a, b, c) in enumerate(CORNERS):
            full = idxs[ci]
            par = full & 1
            w = wxy[a][b] * wt[c]
            gidx_ref[l * 8 + ci] = (
                lax.shift_right_logical(full, 1).astype(jnp.float32))
            w_ref[l * 8 + ci] = jnp.where(par == 1, -w, w)


def _index_call(xt):
    rows = xt.shape[1]
    sb = 64
    return pl.pallas_call(
        _index_body,
        grid=(rows // sb,),
        in_specs=[pl.BlockSpec((3, sb, 128), lambda i: (0, i, 0))],
        out_specs=[pl.BlockSpec((64, sb, 128), lambda i: (0, i, 0)),
                   pl.BlockSpec((64, sb, 128), lambda i: (0, i, 0))],
        out_shape=[jax.ShapeDtypeStruct((64, rows, 128), jnp.float32),
                   jax.ShapeDtypeStruct((64, rows, 128), jnp.float32)],
    )(xt)


def _tr_body(a0_ref, a1_ref, i_ref, oi_ref):
    a0 = a0_ref[...]
    a1 = a1_ref[...]
    parts = []
    for j in range(8):
        t = i_ref[:, j, :].T
        lo = jnp.dot(a0, t, preferred_element_type=jnp.float32,
                     precision=lax.Precision.HIGHEST)
        hi = jnp.dot(a1, t, preferred_element_type=jnp.float32,
                     precision=lax.Precision.HIGHEST)
        parts.append(jnp.concatenate([lo, hi], axis=1))
    oi_ref[...] = jnp.concatenate(parts, axis=0).astype(jnp.int32)


def _transpose_call(gidx_cm3):
    rows = gidx_cm3.shape[1]
    bn = rows * 128
    a0 = np.zeros((64, 128), np.float32)
    a1 = np.zeros((64, 128), np.float32)
    for u in range(64):
        a0[u, 2 * u] = 1.0
        a1[u, 2 * u + 1] = 1.0
    return pl.pallas_call(
        _tr_body,
        grid=(bn // 1024,),
        in_specs=[pl.BlockSpec((64, 128), lambda i: (0, 0)),
                  pl.BlockSpec((64, 128), lambda i: (0, 0)),
                  pl.BlockSpec((64, 8, 128), lambda i: (0, i, 0))],
        out_specs=pl.BlockSpec((512, 128), lambda i: (i, 0)),
        out_shape=jax.ShapeDtypeStruct((bn * 64 // 128, 128), jnp.int32),
    )(jnp.asarray(a0), jnp.asarray(a1), gidx_cm3)


def _gather_call(pairs, idx2d):
    tot = idx2d.shape[0] * 128
    npw = tot // NW
    niter = npw // GCH
    mesh = plsc.VectorSubcoreMesh(core_axis_name="c", subcore_axis_name="s")

    @functools.partial(
        pl.kernel,
        out_type=jax.ShapeDtypeStruct((tot, 8), jnp.float32),
        mesh=mesh,
        compiler_params=pltpu.CompilerParams(use_tc_tiling_on_sc=False),
        scratch_types=[
            pltpu.VMEM((GCH // 128, 128), jnp.int32),
            pltpu.VMEM((GCH, 8), jnp.float32),
            pltpu.SemaphoreType.DMA,
        ],
    )
    def gk(pairs_hbm, idx_hbm, out_hbm, idx_v, rows_v, sem):
        wid = lax.axis_index("s") * NC + lax.axis_index("c")
        base = wid * npw
        kpc = GCH // 128

        @pl.loop(0, niter)
        def _(it):
            off = pl.multiple_of(base + it * GCH, GCH)
            irow = pl.multiple_of(off // 128, kpc)
            pltpu.sync_copy(idx_hbm.at[pl.ds(irow, kpc)], idx_v)
            copies = [
                pltpu.async_copy(pairs_hbm.at[idx_v.at[k]],
                                 rows_v.at[pl.ds(k * 128, 128)], sem)
                for k in range(kpc)
            ]
            for cp in copies:
                cp.wait()
            pltpu.sync_copy(rows_v, out_hbm.at[pl.ds(off, GCH)])

    return gk(pairs, idx2d)


def _reduce_body(f_ref, w_ref, elo_ref, ehi_ref, s_ref, o_ref):
    ws = jnp.concatenate([w_ref[:, j, :].T for j in range(8)], axis=0)
    wlo = jnp.maximum(ws, 0.0)
    whi = jnp.maximum(-ws, 0.0)
    w512 = (jnp.dot(wlo, elo_ref[...], preferred_element_type=jnp.float32)
            + jnp.dot(whi, ehi_ref[...], preferred_element_type=jnp.float32))
    p = f_ref[...] * w512
    o_ref[...] = jnp.dot(p, s_ref[...], preferred_element_type=jnp.float32)


def _reduce_call(feats2, w_pm, elo, ehi, s_mat):
    bn = feats2.shape[0]
    nt = 1024
    return pl.pallas_call(
        _reduce_body,
        grid=(bn // nt,),
        in_specs=[pl.BlockSpec((nt, 512), lambda i: (i, 0)),
                  pl.BlockSpec((64, 8, 128), lambda i: (0, i, 0)),
                  pl.BlockSpec((64, 512), lambda i: (0, 0)),
                  pl.BlockSpec((64, 512), lambda i: (0, 0)),
                  pl.BlockSpec((512, 32), lambda i: (0, 0))],
        out_specs=pl.BlockSpec((nt, 32), lambda i: (i, 0)),
        out_shape=jax.ShapeDtypeStruct((bn, 32), jnp.float32),
    )(feats2, w_pm, elo, ehi, s_mat)


def _const_mats():
    elo = np.zeros((64, 512), np.float32)
    ehi = np.zeros((64, 512), np.float32)
    for c in range(64):
        for f in range(FEAT):
            elo[c, c * 8 + f] = 1.0
            ehi[c, c * 8 + 4 + f] = 1.0
    s = np.kron(np.eye(LEVELS, dtype=np.float32),
                np.kron(np.ones((8, 1), np.float32),
                        np.vstack([np.eye(FEAT, dtype=np.float32),
                                   np.eye(FEAT, dtype=np.float32)])))
    return jnp.asarray(elo), jnp.asarray(ehi), jnp.asarray(s)


def kernel(eventBlock, hashmap):
    b, n, _ = eventBlock.shape
    bn = b * n
    xt = eventBlock.reshape(bn, 3).T.reshape(3, bn // 128, 128)
    gidx_cm3, w_cm3 = _index_call(xt)
    ridx = _transpose_call(gidx_cm3)
    pairs = hashmap.reshape(LEVELS * TABLE // 2, 2 * FEAT)
    feats = _gather_call(pairs, ridx)
    elo, ehi, s_mat = _const_mats()
    out = _reduce_call(feats.reshape(bn, 512), w_cm3, elo, ehi, s_mat)
    return out.reshape(b, n, LEVELS * FEAT)
